# Initial kernel scaffold; baseline (speedup 1.0000x reference)
import jax
import jax.numpy as jnp
from jax.experimental import pallas as pl


def kernel(celebrities, partners, teams, obs_ids, zj, dzj, j_pct, rj, all_feats, theta_w, u_w, phi_w, r_w, beta):
    raise NotImplementedError("write your pallas kernel here")


# speedup vs baseline: 1.9201x; 1.9201x over previous
import jax
import jax.numpy as jnp
from jax import lax
from jax.experimental import pallas as pl
from jax.experimental.pallas import tpu as pltpu
from jax.experimental.pallas import tpu_sc as plsc

EPS = 1e-06
K_VAR = 1.0
LAMBDA_PERF = 1.0

N = 16384
FEAT_DIM = 128

_info = plsc.get_sparse_core_info()
_NC = _info.num_cores
_NS = _info.num_subcores
_NW = _NC * _NS
CHUNK = N // _NW


def _sc_body(teams_hbm, celebs_hbm, partners_hbm, obs_hbm,
             feats_hbm, theta_hbm, u_hbm, r_hbm, phi_hbm,
             ids_hbm, idd_hbm,
             teams_v, celebs_v, partners_v, obs_v,
             feats_v, theta_v, u_v, r_v, phi_v, ids_v, idd_v, sem):
    wid = lax.axis_index("s") * _NC + lax.axis_index("c")
    base = wid * CHUNK

    pltpu.sync_copy(teams_hbm.at[pl.ds(base, CHUNK)], teams_v)
    pltpu.sync_copy(celebs_hbm.at[pl.ds(base, CHUNK)], celebs_v)
    pltpu.sync_copy(partners_hbm.at[pl.ds(base, CHUNK)], partners_v)
    pltpu.sync_copy(obs_hbm.at[pl.ds(base, CHUNK)], obs_v)
    pltpu.sync_copy(phi_hbm.at[0], phi_v)

    pltpu.async_copy(feats_hbm.at[teams_v], feats_v, sem).wait()
    pltpu.async_copy(theta_hbm.at[celebs_v], theta_v, sem).wait()
    pltpu.async_copy(u_hbm.at[partners_v], u_v, sem).wait()
    pltpu.async_copy(r_hbm.at[obs_v], r_v, sem).wait()

    ph = [phi_v[pl.ds(k * 16, 16)] for k in range(FEAT_DIM // 16)]
    lane = lax.iota(jnp.int32, 16)
    onehot = [(lane == j).astype(jnp.float32) for j in range(16)]

    def group(g, carry):
        base16 = g * 16
        acc = theta_v[pl.ds(base16, 16)] + u_v[pl.ds(base16, 16)]
        for j in range(16):
            row = base16 + j
            pp = feats_v[row, pl.ds(0, 16)] * ph[0]
            for k in range(1, FEAT_DIM // 16):
                pp = pp + feats_v[row, pl.ds(k * 16, 16)] * ph[k]
            acc = acc + jnp.sum(pp) * onehot[j]
        ids_v[pl.ds(base16, 16)] = acc
        idd_v[pl.ds(base16, 16)] = acc + r_v[pl.ds(base16, 16)]
        return carry

    lax.fori_loop(0, CHUNK // 16, group, 0)

    pltpu.sync_copy(ids_v, ids_hbm.at[pl.ds(base, CHUNK)])
    pltpu.sync_copy(idd_v, idd_hbm.at[pl.ds(base, CHUNK)])


_sc_gather = pl.kernel(
    _sc_body,
    mesh=plsc.VectorSubcoreMesh(core_axis_name="c", subcore_axis_name="s"),
    compiler_params=pltpu.CompilerParams(needs_layout_passes=False),
    out_type=[jax.ShapeDtypeStruct((N,), jnp.float32),
              jax.ShapeDtypeStruct((N,), jnp.float32)],
    scratch_types=[
        pltpu.VMEM((CHUNK,), jnp.int32),
        pltpu.VMEM((CHUNK,), jnp.int32),
        pltpu.VMEM((CHUNK,), jnp.int32),
        pltpu.VMEM((CHUNK,), jnp.int32),
        pltpu.VMEM((CHUNK, FEAT_DIM), jnp.float32),
        pltpu.VMEM((CHUNK,), jnp.float32),
        pltpu.VMEM((CHUNK,), jnp.float32),
        pltpu.VMEM((CHUNK,), jnp.float32),
        pltpu.VMEM((FEAT_DIM,), jnp.float32),
        pltpu.VMEM((CHUNK,), jnp.float32),
        pltpu.VMEM((CHUNK,), jnp.float32),
        pltpu.SemaphoreType.DMA,
    ],
)


def _tc_body(ids_ref, idd_ref, zj_ref, dzj_ref, jp_ref, beta_ref,
             p_ref, st_ref, alpha_ref):
    ids = ids_ref[...]
    jp = jp_ref[...]
    m_ids = jnp.mean(ids)
    sigma_fan2 = jnp.mean((ids - m_ids) ** 2)
    m_jp = jnp.mean(jp)
    sigma_judge2 = jnp.mean((jp - m_jp) ** 2)
    alpha = sigma_judge2 / (sigma_judge2 + K_VAR * sigma_fan2 + EPS)
    perf = beta_ref[0] * zj_ref[...] + beta_ref[1] * dzj_ref[...]
    eta = (1.0 - alpha) * idd_ref[...] + alpha * LAMBDA_PERF * perf
    mx = jnp.max(eta)
    e = jnp.exp(eta - mx)
    p = e / jnp.sum(e)
    p_ref[...] = p
    st_ref[...] = jp + p
    alpha_ref[...] = jnp.reshape(alpha, (1, 1))


def kernel(celebrities, partners, teams, obs_ids, zj, dzj, j_pct, rj,
           all_feats, theta_w, u_w, phi_w, r_w, beta):
    id_static, id_dyn = _sc_gather(
        teams, celebrities, partners, obs_ids,
        all_feats, theta_w.reshape(-1), u_w.reshape(-1), r_w.reshape(-1),
        phi_w)

    shape2d = (N // 128, 128)
    p2, st2, alpha11 = pl.pallas_call(
        _tc_body,
        out_shape=[
            jax.ShapeDtypeStruct(shape2d, jnp.float32),
            jax.ShapeDtypeStruct(shape2d, jnp.float32),
            jax.ShapeDtypeStruct((1, 1), jnp.float32),
        ],
        in_specs=[
            pl.BlockSpec(memory_space=pltpu.VMEM),
            pl.BlockSpec(memory_space=pltpu.VMEM),
            pl.BlockSpec(memory_space=pltpu.VMEM),
            pl.BlockSpec(memory_space=pltpu.VMEM),
            pl.BlockSpec(memory_space=pltpu.VMEM),
            pl.BlockSpec(memory_space=pltpu.SMEM),
        ],
    )(id_static.reshape(shape2d), id_dyn.reshape(shape2d),
      zj.reshape(shape2d), dzj.reshape(shape2d), j_pct.reshape(shape2d),
      beta)

    return (p2.reshape(N), st2.reshape(N), alpha11.reshape(()), id_static)
